# baseline (speedup 1.0000x reference)
import functools

import jax
import jax.numpy as jnp
from jax import lax
from jax.experimental import pallas as pl
from jax.experimental.pallas import tpu as pltpu
from jax.experimental.pallas import tpu_sc as plsc

N = 10000
NP = 10240
E = 320000
D_IN = 128
D1 = 128
D2 = 64
NC = 2
NS = 16
NT = NC * NS
CH = 128
CPT = 80
EPT = CPT * CH
EP = NT * EPT
DEGW = 16
BR = 256

BOUND = 9728
RPS = BOUND // NS
REM = N - BOUND
REMP = REM + 1
DRPS = NP // NS

_mesh = plsc.VectorSubcoreMesh(
    core_axis_name="c", subcore_axis_name="s", num_cores=NC, num_subcores=NS
)


def _deg_body(dst_hbm, degp_hbm, dstb, ones, zbuf, acc):
    c = lax.axis_index("c")
    s = lax.axis_index("s")
    wid = c * NS + s
    pltpu.sync_copy(dst_hbm.at[wid], dstb)

    @pl.loop(0, CH)
    def _(i):
        ones[i, :] = jnp.full((DEGW,), 1.0, jnp.float32)
        zbuf[i, :] = jnp.zeros((DEGW,), jnp.float32)

    for k in range(DRPS // CH):
        pltpu.sync_copy(zbuf, acc.at[pl.ds(s * DRPS + k * CH, CH)])
    plsc.subcore_barrier()

    @pl.loop(0, CPT)
    def _(j):
        pltpu.sync_copy(ones, acc.at[dstb.at[j]], add=True)

    plsc.subcore_barrier()
    pltpu.sync_copy(acc.at[pl.ds(s * DRPS, DRPS)],
                    degp_hbm.at[c].at[pl.ds(s * DRPS, DRPS)])


_deg_kernel = functools.partial(
    pl.kernel,
    out_type=jax.ShapeDtypeStruct((NC, NP, DEGW), jnp.float32),
    mesh=_mesh,
    scratch_types=[
        pltpu.VMEM((CPT, CH), jnp.int32),
        pltpu.VMEM((CH, DEGW), jnp.float32),
        pltpu.VMEM((CH, DEGW), jnp.float32),
        pltpu.VMEM_SHARED((NP, DEGW), jnp.float32),
    ],
)(_deg_body)


def _agg_body(g_hbm, src_hbm, i1_hbm, srcr_hbm, ir_hbm, pa_hbm, pb_hbm,
              srcb, idxb, r0, r1, accS, sem):
    c = lax.axis_index("c")
    s = lax.axis_index("s")
    wid = c * NS + s

    def zero_rows(zrows):
        for k in range(zrows // CH):
            pltpu.sync_copy(r0, accS.at[pl.ds(s * zrows + k * CH, CH)])
        if zrows % CH:
            pltpu.sync_copy(
                r0.at[pl.ds(0, zrows % CH)],
                accS.at[pl.ds(s * zrows + (zrows // CH) * CH, zrows % CH)])

    def edge_loop():
        @pl.loop(0, CPT)
        def _(j):
            pltpu.sync_copy(g_hbm.at[srcb.at[j]], r0)
            pltpu.sync_copy(r0, accS.at[idxb.at[j]], add=True)

    pltpu.sync_copy(src_hbm.at[wid], srcb)
    pltpu.sync_copy(i1_hbm.at[wid], idxb)

    @pl.loop(0, CH)
    def _(i):
        @pl.loop(0, D1, step=16)
        def _(jj):
            r0[i, pl.ds(jj, 16)] = jnp.zeros((16,), jnp.float32)

    zero_rows(RPS)
    plsc.subcore_barrier()
    edge_loop()
    plsc.subcore_barrier()
    pltpu.sync_copy(accS.at[pl.ds(s * RPS, RPS)],
                    pa_hbm.at[c].at[pl.ds(s * RPS, RPS)])
    plsc.subcore_barrier()

    pltpu.sync_copy(srcr_hbm.at[wid], srcb)
    pltpu.sync_copy(ir_hbm.at[wid], idxb)

    @pl.loop(0, CH)
    def _(i):
        @pl.loop(0, D1, step=16)
        def _(jj):
            r0[i, pl.ds(jj, 16)] = jnp.zeros((16,), jnp.float32)

    zero_rows(33)
    plsc.subcore_barrier()
    edge_loop()
    plsc.subcore_barrier()
    pltpu.sync_copy(accS.at[pl.ds(s * 32, 32)],
                    pb_hbm.at[c].at[pl.ds(s * 32, 32)])


_agg_kernel = functools.partial(
    pl.kernel,
    out_type=[
        jax.ShapeDtypeStruct((NC, BOUND, D1), jnp.float32),
        jax.ShapeDtypeStruct((NC, 512, D1), jnp.float32),
    ],
    mesh=_mesh,
    scratch_types=[
        pltpu.VMEM((CPT, CH), jnp.int32),
        pltpu.VMEM((CPT, CH), jnp.int32),
        pltpu.VMEM((CH, D1), jnp.float32),
        pltpu.VMEM((CH, D1), jnp.float32),
        pltpu.VMEM_SHARED((BOUND, D1), jnp.float32),
        pltpu.SemaphoreType.DMA,
    ],
)(_agg_body)


def _dinv(degp_blk):
    deg = jnp.sum(degp_blk[0] + degp_blk[1], axis=1) * (1.0 / DEGW) + 1.0
    return lax.rsqrt(deg)


def _mm(a, b, contract):
    return lax.dot_general(a, b, (contract, ((), ())),
                           preferred_element_type=jnp.float32,
                           precision=lax.Precision.HIGHEST)


def _merge(pb):
    def body(pbb, ob, sb):
        t = pbb[0] + pbb[1]
        ob[...] = t
        tot = jnp.sum(t, axis=0, keepdims=True)
        sb[...] = jnp.concatenate(
            [tot, jnp.zeros((7, 128), jnp.float32)], axis=0)

    return pl.pallas_call(
        body,
        grid=(1,),
        in_specs=[pl.BlockSpec((NC, NP - BOUND, 128), lambda i: (0, 0, 0))],
        out_specs=[
            pl.BlockSpec((NP - BOUND, 128), lambda i: (0, 0)),
            pl.BlockSpec((8, 128), lambda i: (0, 0)),
        ],
        out_shape=[
            jax.ShapeDtypeStruct((NP - BOUND, 128), jnp.float32),
            jax.ShapeDtypeStruct((8, 128), jnp.float32),
        ],
    )(pb)


def _assemble(i, pab, pbsb, spurb, gb):
    rows = i * BR + lax.broadcasted_iota(jnp.int32, (BR, 1), 0)
    ta = pab[0] + pab[1]
    t = jnp.where(rows < BOUND, ta, pbsb[...])
    t = t - jnp.where(rows == 0, spurb[0:1, :], 0.0)
    return t + gb[...]


def _row_mask(i, val):
    rows = i * BR + lax.broadcasted_iota(jnp.int32, (BR, 1), 0)
    return jnp.where(rows < N, val, 0.0)


_PBS_SPEC = pl.BlockSpec(
    (BR, 128), lambda i: (jnp.where(i < BOUND // BR, 0, i - BOUND // BR), 0))
_PA_SPEC = pl.BlockSpec(
    (NC, BR, D1),
    lambda i: (0, jnp.where(i < BOUND // BR, i, BOUND // BR - 1), 0))


def _tc1(xp, W1, degp):
    def body(xb, w1, dp, ob):
        i = pl.program_id(0)
        dinv = _dinv(dp)
        h = _mm(xb[...], w1[...], ((1,), (1,)))
        ob[...] = _row_mask(i, h * dinv[:, None])

    return pl.pallas_call(
        body,
        grid=(NP // BR,),
        in_specs=[
            pl.BlockSpec((BR, D_IN), lambda i: (i, 0)),
            pl.BlockSpec((D1, D_IN), lambda i: (0, 0)),
            pl.BlockSpec((NC, BR, DEGW), lambda i: (0, i, 0)),
        ],
        out_specs=pl.BlockSpec((BR, D1), lambda i: (i, 0)),
        out_shape=jax.ShapeDtypeStruct((NP, D1), jnp.float32),
    )(xp, W1, degp)


def _tc2(pa, pbs, spur, g1, degp, b1r, W2):
    def body(pab, pbsb, spurb, g1b, dp, b1b, w2, ob):
        i = pl.program_id(0)
        dinv = _dinv(dp)
        t = _assemble(i, pab, pbsb, spurb, g1b)
        h1 = jnp.maximum(t * dinv[:, None] + b1b[...], 0.0)
        h2 = _mm(h1, w2[...], ((1,), (1,)))
        g2 = _row_mask(i, h2 * dinv[:, None])
        ob[...] = jnp.concatenate(
            [g2, jnp.zeros((BR, D1 - D2), jnp.float32)], axis=1)

    return pl.pallas_call(
        body,
        grid=(NP // BR,),
        in_specs=[
            _PA_SPEC,
            _PBS_SPEC,
            pl.BlockSpec((8, 128), lambda i: (0, 0)),
            pl.BlockSpec((BR, D1), lambda i: (i, 0)),
            pl.BlockSpec((NC, BR, DEGW), lambda i: (0, i, 0)),
            pl.BlockSpec((1, D1), lambda i: (0, 0)),
            pl.BlockSpec((D2, D1), lambda i: (0, 0)),
        ],
        out_specs=pl.BlockSpec((BR, D1), lambda i: (i, 0)),
        out_shape=jax.ShapeDtypeStruct((NP, D1), jnp.float32),
    )(pa, pbs, spur, g1, degp, b1r, W2)


def _tc3(qa, qbs, qspur, g2, degp, b2r, wfct, bfcp):
    def body(pab, pbsb, spurb, g2b, dp, b2b, wf, bf, ob):
        i = pl.program_id(0)
        dinv = _dinv(dp)
        t = _assemble(i, pab, pbsb, spurb, g2b)[:, :D2]
        h2 = jnp.maximum(t * dinv[:, None] + b2b[...], 0.0)
        ob[...] = _mm(h2, wf[...], ((1,), (0,))) + bf[...]

    return pl.pallas_call(
        body,
        grid=(NP // BR,),
        in_specs=[
            _PA_SPEC,
            _PBS_SPEC,
            pl.BlockSpec((8, 128), lambda i: (0, 0)),
            pl.BlockSpec((BR, D1), lambda i: (i, 0)),
            pl.BlockSpec((NC, BR, DEGW), lambda i: (0, i, 0)),
            pl.BlockSpec((1, D2), lambda i: (0, 0)),
            pl.BlockSpec((D2, 128), lambda i: (0, 0)),
            pl.BlockSpec((1, 128), lambda i: (0, 0)),
        ],
        out_specs=pl.BlockSpec((BR, 128), lambda i: (i, 0)),
        out_shape=jax.ShapeDtypeStruct((NP, 128), jnp.float32),
    )(qa, qbs, qspur, g2, degp, b2r, wfct, bfcp)


def kernel(x, edge_index, W1, b1, W2, b2, Wfc, bfc):
    f32 = jnp.float32
    src = edge_index[0].astype(jnp.int32)
    dst = edge_index[1].astype(jnp.int32)
    srcp = jnp.concatenate([src, jnp.full((EP - E,), NP - 1, jnp.int32)])
    dstp = jnp.concatenate([dst, jnp.zeros((EP - E,), jnp.int32)])
    src2 = srcp.reshape(NT, CPT, CH)
    dst3 = jnp.concatenate(
        [dst, jnp.full((EP - E,), NP - 1, jnp.int32)]).reshape(NT, CPT, CH)
    rem_mask = dstp >= BOUND
    idx1 = jnp.where(rem_mask, 0, dstp).reshape(NT, CPT, CH)
    idxr = jnp.where(rem_mask, dstp - BOUND, 512).reshape(NT, CPT, CH)
    srcr = jnp.where(rem_mask, srcp, NP - 1).reshape(NT, CPT, CH)
    xp = jnp.pad(x, ((0, NP - N), (0, 0)))
    wfct = jnp.zeros((D2, 128), f32).at[:, :6].set(Wfc.T)
    bfcp = jnp.zeros((1, 128), f32).at[0, :6].set(bfc)

    degp = _deg_kernel(dst3)
    g1 = _tc1(xp, W1, degp)
    pa, pb = _agg_kernel(g1, src2, idx1, srcr, idxr)
    pbs, spur = _merge(pb)
    g2 = _tc2(pa, pbs, spur, g1, degp, b1.reshape(1, D1), W2)
    qa, qb = _agg_kernel(g2, src2, idx1, srcr, idxr)
    qbs, qspur = _merge(qb)
    outf = _tc3(qa, qbs, qspur, g2, degp,
                b2.reshape(1, D2), wfct, bfcp)
    return outf[:N, :6]
